# Initial kernel scaffold; baseline (speedup 1.0000x reference)
import jax
import jax.numpy as jnp
from jax.experimental import pallas as pl


def kernel(x, ft_w0, ft_b0, ft_w1, ft_b1, g1_wl, g1_wr, g1_a, g1_b, g2_wl, g2_wr, g2_a, g2_b, g3_wl, g3_wr, g3_a, g3_b, g4_wl, g4_wr, g4_a, g4_b, mlp_w1, mlp_b1, mlp_w2, mlp_b2, mlp_w3, mlp_b3):
    raise NotImplementedError("write your pallas kernel here")


# speedup vs baseline: 1.8216x; 1.8216x over previous
import functools

import jax
import jax.numpy as jnp
from jax import lax
from jax.experimental import pallas as pl
from jax.experimental.pallas import tpu as pltpu
from jax.experimental.pallas import tpu_sc as plsc

N = 4096
K = 16
DIAG = 1e9
TAKEN = 2e9


def _stem_body(xa_ref, xb_ref, w0_ref, b0_ref, w1_ref, b1_ref, f0_ref, f1_ref):
    f0_ref[...] = (
        jnp.dot(xa_ref[...], w0_ref[...], preferred_element_type=jnp.float32)
        + b0_ref[...]
    )
    f1_ref[...] = (
        jnp.dot(xb_ref[...], w1_ref[...], preferred_element_type=jnp.float32)
        + b1_ref[...]
    )


def _stem(xa, xb, w0, b0, w1, b1):
    RB = 1024
    grid = N // RB
    return pl.pallas_call(
        _stem_body,
        grid=(grid,),
        in_specs=[
            pl.BlockSpec((RB, 3), lambda i: (i, 0)),
            pl.BlockSpec((RB, 6), lambda i: (i, 0)),
            pl.BlockSpec((3, 256), lambda i: (0, 0)),
            pl.BlockSpec((1, 256), lambda i: (0, 0)),
            pl.BlockSpec((6, 256), lambda i: (0, 0)),
            pl.BlockSpec((1, 256), lambda i: (0, 0)),
        ],
        out_specs=[
            pl.BlockSpec((RB, 256), lambda i: (i, 0)),
            pl.BlockSpec((RB, 256), lambda i: (i, 0)),
        ],
        out_shape=[
            jax.ShapeDtypeStruct((N, 256), jnp.float32),
            jax.ShapeDtypeStruct((N, 256), jnp.float32),
        ],
    )(xa, xb, w0, b0.reshape(1, 256), w1, b1.reshape(1, 256))


def _knn_body(t_blk_ref, t_all_ref, idx_ref, dist_ref):
    rb = pl.program_id(0)
    RB = t_blk_ref.shape[0]
    d = t_blk_ref.shape[1]
    t_blk = t_blk_ref[...]
    sq_blk = jnp.sum(t_blk * t_blk, axis=1)[:, None]

    CT = 1024
    for ct in range(N // CT):
        t_tile = t_all_ref[pl.ds(ct * CT, CT), :]
        sq_tile = jnp.sum(t_tile * t_tile, axis=1)[None, :]
        prod = lax.dot_general(
            t_blk, t_tile, (((1,), (1,)), ((), ())),
            preferred_element_type=jnp.float32,
        )
        dist_ref[:, pl.ds(ct * CT, CT)] = sq_blk + sq_tile - 2.0 * prod

    def per_chunk(rc, carry):
        d8 = dist_ref[pl.ds(rc * 8, 8), :]
        col8 = lax.broadcasted_iota(jnp.int32, (8, N), 1)
        row8 = rb * RB + rc * 8 + lax.broadcasted_iota(jnp.int32, (8, N), 0)
        d8 = jnp.where(col8 == row8, DIAG, d8)
        cols = []
        for _ in range(K):
            m = jnp.min(d8, axis=1, keepdims=True)
            j = jnp.min(jnp.where(d8 == m, col8, N), axis=1, keepdims=True)
            cols.append(j)
            d8 = jnp.where(col8 == j, TAKEN, d8)
        idx_ref[pl.ds(rc * 8, 8), :] = jnp.concatenate(cols, axis=1)
        return carry

    lax.fori_loop(0, RB // 8, per_chunk, 0)


def _knn(t):
    d = t.shape[1]
    RB = 256
    grid = N // RB
    return pl.pallas_call(
        _knn_body,
        grid=(grid,),
        in_specs=[
            pl.BlockSpec((RB, d), lambda i: (i, 0)),
            pl.BlockSpec((N, d), lambda i: (0, 0)),
        ],
        out_specs=pl.BlockSpec((RB, K), lambda i: (i, 0)),
        out_shape=jax.ShapeDtypeStruct((N, K), jnp.int32),
        scratch_shapes=[pltpu.VMEM((RB, N), jnp.float32)],
    )(t, t)


def _proj_body(t_ref, wl_ref, wr_ref, hl_ref, hr_ref):
    t = t_ref[...]
    hl_ref[...] = jnp.dot(t, wl_ref[...], preferred_element_type=jnp.float32)
    hr_ref[...] = jnp.dot(t, wr_ref[...], preferred_element_type=jnp.float32)


def _proj(t, wl, wr):
    di, do = wl.shape
    RB = 512
    grid = N // RB
    return pl.pallas_call(
        _proj_body,
        grid=(grid,),
        in_specs=[
            pl.BlockSpec((RB, di), lambda i: (i, 0)),
            pl.BlockSpec((di, do), lambda i: (0, 0)),
            pl.BlockSpec((di, do), lambda i: (0, 0)),
        ],
        out_specs=[
            pl.BlockSpec((RB, do), lambda i: (i, 0)),
            pl.BlockSpec((RB, do), lambda i: (i, 0)),
        ],
        out_shape=[
            jax.ShapeDtypeStruct((N, do), jnp.float32),
            jax.ShapeDtypeStruct((N, do), jnp.float32),
        ],
    )(t, wl, wr)


def _sc_gather(table, idx_flat):
    D = table.shape[1]
    B = idx_flat.shape[0]
    NW = 32
    b_per_w = B // NW
    C = 128
    n_chunks = b_per_w // C
    mesh = plsc.VectorSubcoreMesh(core_axis_name="c", subcore_axis_name="s")

    @functools.partial(
        pl.kernel,
        mesh=mesh,
        out_type=jax.ShapeDtypeStruct((B, D), jnp.float32),
        scratch_types=[
            pltpu.VMEM((b_per_w,), jnp.int32),
            pltpu.VMEM((C, D), jnp.float32),
            pltpu.SemaphoreType.DMA,
        ],
    )
    def k(table_hbm, idx_hbm, out_hbm, idx_v, rows_v, sem):
        wid = lax.axis_index("s") * 2 + lax.axis_index("c")
        base = wid * b_per_w
        pltpu.sync_copy(idx_hbm.at[pl.ds(base, b_per_w)], idx_v)

        def body(c, carry):
            off = c * C
            pltpu.async_copy(
                table_hbm.at[idx_v.at[pl.ds(off, C)]], rows_v, sem
            ).wait()
            pltpu.sync_copy(rows_v, out_hbm.at[pl.ds(base + off, C)])
            return carry

        lax.fori_loop(0, n_chunks, body, 0)

    return k(table, idx_flat)


def _edge_body(hl_ref, hn_ref, a_ref, b_ref, out_ref):
    hl = hl_ref[...]
    hn = hn_ref[...]
    a = a_ref[...]
    z = hl[:, None, :] + hn
    z = jnp.where(z > 0, z, 0.2 * z)
    e = jnp.sum(z * a[None, :, :], axis=-1)
    e = e - jnp.max(e, axis=1, keepdims=True)
    w = jnp.exp(e)
    alpha = w / jnp.sum(w, axis=1, keepdims=True)
    out_ref[...] = jnp.sum(alpha[:, :, None] * hn, axis=1) + b_ref[...]


def _edge(hl, hn3, a, b):
    D = hl.shape[1]
    NB = 128
    grid = N // NB
    return pl.pallas_call(
        _edge_body,
        grid=(grid,),
        in_specs=[
            pl.BlockSpec((NB, D), lambda i: (i, 0)),
            pl.BlockSpec((NB, K, D), lambda i: (i, 0, 0)),
            pl.BlockSpec((1, D), lambda i: (0, 0)),
            pl.BlockSpec((1, D), lambda i: (0, 0)),
        ],
        out_specs=pl.BlockSpec((NB, D), lambda i: (i, 0)),
        out_shape=jax.ShapeDtypeStruct((N, D), jnp.float32),
    )(hl, hn3, a.reshape(1, D), b.reshape(1, D))


def _mean_body(a_ref, b_ref, o_ref):
    o_ref[...] = 0.5 * (a_ref[...] + b_ref[...])


def _mean2(a, b):
    RB = 1024
    return pl.pallas_call(
        _mean_body,
        grid=(N // RB,),
        in_specs=[
            pl.BlockSpec((RB, 256), lambda i: (i, 0)),
            pl.BlockSpec((RB, 256), lambda i: (i, 0)),
        ],
        out_specs=pl.BlockSpec((RB, 256), lambda i: (i, 0)),
        out_shape=jax.ShapeDtypeStruct((N, 256), jnp.float32),
    )(a, b)


def _mlp_body(y1_ref, y2_ref, y3_ref, y4_ref, w1_ref, b1_ref, w2_ref, b2_ref,
              w3_ref, b3_ref, out_ref):
    acc = jnp.dot(y1_ref[...], w1_ref[0:256, :],
                  preferred_element_type=jnp.float32)
    acc += jnp.dot(y2_ref[...], w1_ref[256:768, :],
                   preferred_element_type=jnp.float32)
    acc += jnp.dot(y3_ref[...], w1_ref[768:1280, :],
                   preferred_element_type=jnp.float32)
    acc += jnp.dot(y4_ref[...], w1_ref[1280:1792, :],
                   preferred_element_type=jnp.float32)
    h1 = jnp.maximum(acc + b1_ref[...], 0.0)
    h2 = jnp.maximum(
        jnp.dot(h1, w2_ref[...], preferred_element_type=jnp.float32)
        + b2_ref[...], 0.0)
    out_ref[...] = (
        jnp.dot(h2, w3_ref[...], preferred_element_type=jnp.float32)
        + b3_ref[...]
    )


def _mlp(y1, y2, y3, y4, w1, b1, w2, b2, w3, b3):
    RB = 512
    return pl.pallas_call(
        _mlp_body,
        grid=(N // RB,),
        in_specs=[
            pl.BlockSpec((RB, 256), lambda i: (i, 0)),
            pl.BlockSpec((RB, 512), lambda i: (i, 0)),
            pl.BlockSpec((RB, 512), lambda i: (i, 0)),
            pl.BlockSpec((RB, 512), lambda i: (i, 0)),
            pl.BlockSpec((1792, 256), lambda i: (0, 0)),
            pl.BlockSpec((1, 256), lambda i: (0, 0)),
            pl.BlockSpec((256, 64), lambda i: (0, 0)),
            pl.BlockSpec((1, 64), lambda i: (0, 0)),
            pl.BlockSpec((64, 3), lambda i: (0, 0)),
            pl.BlockSpec((1, 3), lambda i: (0, 0)),
        ],
        out_specs=pl.BlockSpec((RB, 3), lambda i: (i, 0)),
        out_shape=jax.ShapeDtypeStruct((N, 3), jnp.float32),
    )(y1, y2, y3, y4, w1, b1.reshape(1, 256), w2, b2.reshape(1, 64),
      w3, b3.reshape(1, 3))


def _gat_unit(t, wl, wr, a, b):
    do = wl.shape[1]
    idx = _knn(t)
    hl, hr = _proj(t, wl, wr)
    hn = _sc_gather(hr, idx.reshape(N * K))
    return _edge(hl, hn.reshape(N, K, do), a, b)


def kernel(x, ft_w0, ft_b0, ft_w1, ft_b1, g1_wl, g1_wr, g1_a, g1_b,
           g2_wl, g2_wr, g2_a, g2_b, g3_wl, g3_wr, g3_a, g3_b,
           g4_wl, g4_wr, g4_a, g4_b, mlp_w1, mlp_b1, mlp_w2, mlp_b2,
           mlp_w3, mlp_b3):
    xa = x[:, 0:3]
    xb = x[:, 3:9]
    f0, f1 = _stem(xa, xb, ft_w0, ft_b0, ft_w1, ft_b1)
    y0 = _gat_unit(f0, g1_wl, g1_wr, g1_a, g1_b)
    y1 = _gat_unit(f1, g1_wl, g1_wr, g1_a, g1_b)
    y = _mean2(y0, y1)
    y2 = _gat_unit(y, g2_wl, g2_wr, g2_a, g2_b)
    y3 = _gat_unit(y2, g3_wl, g3_wr, g3_a, g3_b)
    y4 = _gat_unit(y3, g4_wl, g4_wr, g4_a, g4_b)
    return _mlp(y, y2, y3, y4, mlp_w1, mlp_b1, mlp_w2, mlp_b2, mlp_w3, mlp_b3)


# speedup vs baseline: 5.4386x; 2.9857x over previous
import functools

import jax
import jax.numpy as jnp
from jax import lax
from jax.experimental import pallas as pl
from jax.experimental.pallas import tpu as pltpu
from jax.experimental.pallas import tpu_sc as plsc

N = 4096
K = 16
BIGF = 3.0e38


def _stem_body(xa_ref, xb_ref, w0_ref, b0_ref, w1_ref, b1_ref, f0_ref, f1_ref):
    f0_ref[...] = (
        jnp.dot(xa_ref[...], w0_ref[...], preferred_element_type=jnp.float32)
        + b0_ref[...]
    )
    f1_ref[...] = (
        jnp.dot(xb_ref[...], w1_ref[...], preferred_element_type=jnp.float32)
        + b1_ref[...]
    )


def _stem(xa, xb, w0, b0, w1, b1):
    RB = 1024
    grid = N // RB
    return pl.pallas_call(
        _stem_body,
        grid=(grid,),
        in_specs=[
            pl.BlockSpec((RB, 3), lambda i: (i, 0)),
            pl.BlockSpec((RB, 6), lambda i: (i, 0)),
            pl.BlockSpec((3, 256), lambda i: (0, 0)),
            pl.BlockSpec((1, 256), lambda i: (0, 0)),
            pl.BlockSpec((6, 256), lambda i: (0, 0)),
            pl.BlockSpec((1, 256), lambda i: (0, 0)),
        ],
        out_specs=[
            pl.BlockSpec((RB, 256), lambda i: (i, 0)),
            pl.BlockSpec((RB, 256), lambda i: (i, 0)),
        ],
        out_shape=[
            jax.ShapeDtypeStruct((N, 256), jnp.float32),
            jax.ShapeDtypeStruct((N, 256), jnp.float32),
        ],
    )(xa, xb, w0, b0.reshape(1, 256), w1, b1.reshape(1, 256))


def _knn_body(t_blk_ref, t_all_ref, idx_ref, dist_ref):
    rb = pl.program_id(0)
    RB = t_blk_ref.shape[0]
    d = t_blk_ref.shape[1]
    t_blk = t_blk_ref[...]
    sq_blk = jnp.sum(t_blk * t_blk, axis=1)[:, None]

    CT = 1024
    for ct in range(N // CT):
        t_tile = t_all_ref[pl.ds(ct * CT, CT), :]
        sq_tile = jnp.sum(t_tile * t_tile, axis=1)[None, :]
        prod = lax.dot_general(
            t_blk, t_tile, (((1,), (1,)), ((), ())),
            preferred_element_type=jnp.float32,
        )
        dist_ref[:, pl.ds(ct * CT, CT)] = sq_blk + sq_tile - 2.0 * prod

    R = 32
    NV = N // 128
    lane = lax.broadcasted_iota(jnp.int32, (R, 128), 1)

    def per_chunk(rc, carry):
        base = rc * R
        excl = rb * RB + base + lax.broadcasted_iota(jnp.int32, (R, 128), 0)
        cols = []
        for _ in range(K):
            m = jnp.full((R, 128), BIGF, jnp.float32)
            w = jnp.zeros((R, 128), jnp.int32)
            for v in range(NV):
                dv = dist_ref[pl.ds(base, R), v * 128:(v + 1) * 128]
                cid = lane + (v * 128)
                dvx = jnp.where(cid == excl, BIGF, dv)
                dist_ref[pl.ds(base, R), v * 128:(v + 1) * 128] = dvx
                better = dvx < m
                w = jnp.where(better, cid, w)
                m = jnp.where(better, dvx, m)
            mv = jnp.min(m, axis=1, keepdims=True)
            j = jnp.min(jnp.where(m == mv, w, jnp.int32(N)), axis=1,
                        keepdims=True)
            cols.append(j)
            excl = jnp.broadcast_to(j, (R, 128))
        idx_ref[pl.ds(base, R), :] = jnp.concatenate(cols, axis=1)
        return carry

    lax.fori_loop(0, RB // R, per_chunk, 0)


def _knn(t):
    d = t.shape[1]
    RB = 256
    grid = N // RB
    return pl.pallas_call(
        _knn_body,
        grid=(grid,),
        in_specs=[
            pl.BlockSpec((RB, d), lambda i: (i, 0)),
            pl.BlockSpec((N, d), lambda i: (0, 0)),
        ],
        out_specs=pl.BlockSpec((RB, K), lambda i: (i, 0)),
        out_shape=jax.ShapeDtypeStruct((N, K), jnp.int32),
        scratch_shapes=[pltpu.VMEM((RB, N), jnp.float32)],
    )(t, t)


def _proj_body(t_ref, wl_ref, wr_ref, hl_ref, hr_ref):
    t = t_ref[...]
    hl_ref[...] = jnp.dot(t, wl_ref[...], preferred_element_type=jnp.float32)
    hr_ref[...] = jnp.dot(t, wr_ref[...], preferred_element_type=jnp.float32)


def _proj(t, wl, wr):
    di, do = wl.shape
    RB = 512
    grid = N // RB
    return pl.pallas_call(
        _proj_body,
        grid=(grid,),
        in_specs=[
            pl.BlockSpec((RB, di), lambda i: (i, 0)),
            pl.BlockSpec((di, do), lambda i: (0, 0)),
            pl.BlockSpec((di, do), lambda i: (0, 0)),
        ],
        out_specs=[
            pl.BlockSpec((RB, do), lambda i: (i, 0)),
            pl.BlockSpec((RB, do), lambda i: (i, 0)),
        ],
        out_shape=[
            jax.ShapeDtypeStruct((N, do), jnp.float32),
            jax.ShapeDtypeStruct((N, do), jnp.float32),
        ],
    )(t, wl, wr)


def _sc_gather(table, idx_flat):
    D = table.shape[1]
    B = idx_flat.shape[0]
    NW = 32
    b_per_w = B // NW
    C = 128
    n_chunks = b_per_w // C
    mesh = plsc.VectorSubcoreMesh(core_axis_name="c", subcore_axis_name="s")

    @functools.partial(
        pl.kernel,
        mesh=mesh,
        out_type=jax.ShapeDtypeStruct((B, D), jnp.float32),
        scratch_types=[
            pltpu.VMEM((b_per_w,), jnp.int32),
            pltpu.VMEM((C, D), jnp.float32),
            pltpu.SemaphoreType.DMA,
        ],
    )
    def k(table_hbm, idx_hbm, out_hbm, idx_v, rows_v, sem):
        wid = lax.axis_index("s") * 2 + lax.axis_index("c")
        base = wid * b_per_w
        pltpu.sync_copy(idx_hbm.at[pl.ds(base, b_per_w)], idx_v)

        def body(c, carry):
            off = c * C
            pltpu.async_copy(
                table_hbm.at[idx_v.at[pl.ds(off, C)]], rows_v, sem
            ).wait()
            pltpu.sync_copy(rows_v, out_hbm.at[pl.ds(base + off, C)])
            return carry

        lax.fori_loop(0, n_chunks, body, 0)

    return k(table, idx_flat)


def _edge_body(hl_ref, hn_ref, a_ref, b_ref, out_ref):
    hl = hl_ref[...]
    hn = hn_ref[...]
    a = a_ref[...]
    z = hl[:, None, :] + hn
    z = jnp.where(z > 0, z, 0.2 * z)
    e = jnp.sum(z * a[None, :, :], axis=-1)
    e = e - jnp.max(e, axis=1, keepdims=True)
    w = jnp.exp(e)
    alpha = w / jnp.sum(w, axis=1, keepdims=True)
    out_ref[...] = jnp.sum(alpha[:, :, None] * hn, axis=1) + b_ref[...]


def _edge(hl, hn3, a, b):
    D = hl.shape[1]
    NB = 128
    grid = N // NB
    return pl.pallas_call(
        _edge_body,
        grid=(grid,),
        in_specs=[
            pl.BlockSpec((NB, D), lambda i: (i, 0)),
            pl.BlockSpec((NB, K, D), lambda i: (i, 0, 0)),
            pl.BlockSpec((1, D), lambda i: (0, 0)),
            pl.BlockSpec((1, D), lambda i: (0, 0)),
        ],
        out_specs=pl.BlockSpec((NB, D), lambda i: (i, 0)),
        out_shape=jax.ShapeDtypeStruct((N, D), jnp.float32),
    )(hl, hn3, a.reshape(1, D), b.reshape(1, D))


def _mean_body(a_ref, b_ref, o_ref):
    o_ref[...] = 0.5 * (a_ref[...] + b_ref[...])


def _mean2(a, b):
    RB = 1024
    return pl.pallas_call(
        _mean_body,
        grid=(N // RB,),
        in_specs=[
            pl.BlockSpec((RB, 256), lambda i: (i, 0)),
            pl.BlockSpec((RB, 256), lambda i: (i, 0)),
        ],
        out_specs=pl.BlockSpec((RB, 256), lambda i: (i, 0)),
        out_shape=jax.ShapeDtypeStruct((N, 256), jnp.float32),
    )(a, b)


def _mlp_body(y1_ref, y2_ref, y3_ref, y4_ref, w1_ref, b1_ref, w2_ref, b2_ref,
              w3_ref, b3_ref, out_ref):
    acc = jnp.dot(y1_ref[...], w1_ref[0:256, :],
                  preferred_element_type=jnp.float32)
    acc += jnp.dot(y2_ref[...], w1_ref[256:768, :],
                   preferred_element_type=jnp.float32)
    acc += jnp.dot(y3_ref[...], w1_ref[768:1280, :],
                   preferred_element_type=jnp.float32)
    acc += jnp.dot(y4_ref[...], w1_ref[1280:1792, :],
                   preferred_element_type=jnp.float32)
    h1 = jnp.maximum(acc + b1_ref[...], 0.0)
    h2 = jnp.maximum(
        jnp.dot(h1, w2_ref[...], preferred_element_type=jnp.float32)
        + b2_ref[...], 0.0)
    out_ref[...] = (
        jnp.dot(h2, w3_ref[...], preferred_element_type=jnp.float32)
        + b3_ref[...]
    )


def _mlp(y1, y2, y3, y4, w1, b1, w2, b2, w3, b3):
    RB = 512
    return pl.pallas_call(
        _mlp_body,
        grid=(N // RB,),
        in_specs=[
            pl.BlockSpec((RB, 256), lambda i: (i, 0)),
            pl.BlockSpec((RB, 512), lambda i: (i, 0)),
            pl.BlockSpec((RB, 512), lambda i: (i, 0)),
            pl.BlockSpec((RB, 512), lambda i: (i, 0)),
            pl.BlockSpec((1792, 256), lambda i: (0, 0)),
            pl.BlockSpec((1, 256), lambda i: (0, 0)),
            pl.BlockSpec((256, 64), lambda i: (0, 0)),
            pl.BlockSpec((1, 64), lambda i: (0, 0)),
            pl.BlockSpec((64, 3), lambda i: (0, 0)),
            pl.BlockSpec((1, 3), lambda i: (0, 0)),
        ],
        out_specs=pl.BlockSpec((RB, 3), lambda i: (i, 0)),
        out_shape=jax.ShapeDtypeStruct((N, 3), jnp.float32),
    )(y1, y2, y3, y4, w1, b1.reshape(1, 256), w2, b2.reshape(1, 64),
      w3, b3.reshape(1, 3))


def _gat_unit(t, wl, wr, a, b):
    do = wl.shape[1]
    idx = _knn(t)
    hl, hr = _proj(t, wl, wr)
    hn = _sc_gather(hr, idx.reshape(N * K))
    return _edge(hl, hn.reshape(N, K, do), a, b)


def kernel(x, ft_w0, ft_b0, ft_w1, ft_b1, g1_wl, g1_wr, g1_a, g1_b,
           g2_wl, g2_wr, g2_a, g2_b, g3_wl, g3_wr, g3_a, g3_b,
           g4_wl, g4_wr, g4_a, g4_b, mlp_w1, mlp_b1, mlp_w2, mlp_b2,
           mlp_w3, mlp_b3):
    xa = x[:, 0:3]
    xb = x[:, 3:9]
    f0, f1 = _stem(xa, xb, ft_w0, ft_b0, ft_w1, ft_b1)
    y0 = _gat_unit(f0, g1_wl, g1_wr, g1_a, g1_b)
    y1 = _gat_unit(f1, g1_wl, g1_wr, g1_a, g1_b)
    y = _mean2(y0, y1)
    y2 = _gat_unit(y, g2_wl, g2_wr, g2_a, g2_b)
    y3 = _gat_unit(y2, g3_wl, g3_wr, g3_a, g3_b)
    y4 = _gat_unit(y3, g4_wl, g4_wr, g4_a, g4_b)
    return _mlp(y, y2, y3, y4, mlp_w1, mlp_b1, mlp_w2, mlp_b2, mlp_w3, mlp_b3)


# speedup vs baseline: 5.8592x; 1.0773x over previous
import functools

import jax
import jax.numpy as jnp
from jax import lax
from jax.experimental import pallas as pl
from jax.experimental.pallas import tpu as pltpu
from jax.experimental.pallas import tpu_sc as plsc

N = 4096
K = 16
BIGF = 3.0e38


def _stem_body(xa_ref, xb_ref, w0_ref, b0_ref, w1_ref, b1_ref, f0_ref, f1_ref):
    f0_ref[...] = (
        jnp.dot(xa_ref[...], w0_ref[...], preferred_element_type=jnp.float32)
        + b0_ref[...]
    )
    f1_ref[...] = (
        jnp.dot(xb_ref[...], w1_ref[...], preferred_element_type=jnp.float32)
        + b1_ref[...]
    )


def _stem(xa, xb, w0, b0, w1, b1):
    RB = 1024
    grid = N // RB
    return pl.pallas_call(
        _stem_body,
        grid=(grid,),
        in_specs=[
            pl.BlockSpec((RB, 3), lambda i: (i, 0)),
            pl.BlockSpec((RB, 6), lambda i: (i, 0)),
            pl.BlockSpec((3, 256), lambda i: (0, 0)),
            pl.BlockSpec((1, 256), lambda i: (0, 0)),
            pl.BlockSpec((6, 256), lambda i: (0, 0)),
            pl.BlockSpec((1, 256), lambda i: (0, 0)),
        ],
        out_specs=[
            pl.BlockSpec((RB, 256), lambda i: (i, 0)),
            pl.BlockSpec((RB, 256), lambda i: (i, 0)),
        ],
        out_shape=[
            jax.ShapeDtypeStruct((N, 256), jnp.float32),
            jax.ShapeDtypeStruct((N, 256), jnp.float32),
        ],
    )(xa, xb, w0, b0.reshape(1, 256), w1, b1.reshape(1, 256))


def _dist_body(t_blk_ref, t_all_ref, dist_ref):
    rb = pl.program_id(0)
    RB = t_blk_ref.shape[0]
    t_blk = t_blk_ref[...]
    sq_blk = jnp.sum(t_blk * t_blk, axis=1)[:, None]

    CT = 1024
    for ct in range(N // CT):
        t_tile = t_all_ref[pl.ds(ct * CT, CT), :]
        sq_tile = jnp.sum(t_tile * t_tile, axis=1)[None, :]
        prod = lax.dot_general(
            t_blk, t_tile, (((1,), (1,)), ((), ())),
            preferred_element_type=jnp.float32,
        )
        d2 = sq_blk + sq_tile - 2.0 * prod
        col = ct * CT + lax.broadcasted_iota(jnp.int32, (RB, CT), 1)
        row = rb * RB + lax.broadcasted_iota(jnp.int32, (RB, CT), 0)
        dist_ref[:, pl.ds(ct * CT, CT)] = jnp.where(col == row, BIGF, d2)


def _dist(t):
    d = t.shape[1]
    RB = 256
    return pl.pallas_call(
        _dist_body,
        grid=(N // RB,),
        in_specs=[
            pl.BlockSpec((RB, d), lambda i: (i, 0)),
            pl.BlockSpec((N, d), lambda i: (0, 0)),
        ],
        out_specs=pl.BlockSpec((RB, N), lambda i: (i, 0)),
        out_shape=jax.ShapeDtypeStruct((N, N), jnp.float32),
    )(t, t)


def _sc_topk(dist):
    NW = 32
    RW = N // NW
    RB8 = 8
    NACC = 8
    CC = N // (16 * NACC)
    mesh = plsc.VectorSubcoreMesh(core_axis_name="c", subcore_axis_name="s")

    @functools.partial(
        pl.kernel,
        mesh=mesh,
        out_type=jax.ShapeDtypeStruct((N, K), jnp.int32),
        scratch_types=[
            pltpu.VMEM((RB8, N), jnp.float32),
            pltpu.VMEM((RB8, K), jnp.int32),
            pltpu.VMEM((16,), jnp.int32),
        ],
    )
    def k(dist_hbm, idx_hbm, d_v, o_v, sci_v):
        wid = lax.axis_index("s") * 2 + lax.axis_index("c")
        base = wid * RW
        lane = lax.broadcasted_iota(jnp.int32, (16,), 0)
        bigf = jnp.full((16,), BIGF, jnp.float32)
        bigi = jnp.full((16,), N, jnp.int32)

        xors = [(lane ^ st).reshape(16, 1) for st in (8, 4, 2, 1)]
        _dnums = lax.GatherDimensionNumbers(
            offset_dims=(), collapsed_slice_dims=(0,), start_index_map=(0,))

        def bmin(x):
            for xi in xors:
                perm = lax.gather(
                    x, xi, _dnums, (1,),
                    mode=lax.GatherScatterMode.PROMISE_IN_BOUNDS)
                x = jnp.minimum(x, perm)
            return x

        def batch_body(bb, carry0):
            row0 = base + bb * RB8
            pltpu.sync_copy(dist_hbm.at[pl.ds(row0, RB8)], d_v)

            def row_body(r, carry1):
                def tb(cc, mw):
                    ms, ws = list(mw[0]), list(mw[1])
                    for jj in range(NACC):
                        c = cc * NACC + jj
                        dv = d_v[r, pl.ds(c * 16, 16)]
                        cid = lane + c * 16
                        better = dv < ms[jj]
                        ws[jj] = jnp.where(better, cid, ws[jj])
                        ms[jj] = jnp.where(better, dv, ms[jj])
                    return (tuple(ms), tuple(ws))

                ms, ws = lax.fori_loop(
                    0, CC, tb,
                    (tuple(bigf for _ in range(NACC)),
                     tuple(bigi for _ in range(NACC))))
                ms, ws = list(ms), list(ws)

                outv = jnp.zeros((16,), jnp.int32)
                for kk in range(K):
                    t01 = jnp.minimum(ms[0], ms[1])
                    t23 = jnp.minimum(ms[2], ms[3])
                    t45 = jnp.minimum(ms[4], ms[5])
                    t67 = jnp.minimum(ms[6], ms[7])
                    tmin = jnp.minimum(jnp.minimum(t01, t23),
                                       jnp.minimum(t45, t67))
                    mv = bmin(tmin)
                    wc = bigi
                    for jj in range(NACC):
                        wc = jnp.where(ms[jj] == mv,
                                       jnp.minimum(wc, ws[jj]), wc)
                    j = bmin(wc)
                    outv = jnp.where(lane == kk, j, outv)
                    js = j[0]
                    cj = js >> 4
                    lstar = j & 15
                    lmask = lane == lstar
                    dvj = d_v[r, pl.ds(cj * 16, 16)]
                    d_v[r, pl.ds(cj * 16, 16)] = jnp.where(lmask, bigf, dvj)
                    if kk == K - 1:
                        continue
                    jsel = (j >> 4) & (NACC - 1)
                    gsel = (js >> 4) & (NACC - 1)
                    cbase = (jsel << 4) + lstar
                    macc = bigf
                    wacc = bigi
                    for cc in range(32):
                        v = d_v[r, pl.ds(cc * 128 + gsel * 16, 16)]
                        vs = jnp.where(lmask, v, BIGF)
                        cid = cbase + cc * 128
                        better = vs < macc
                        wacc = jnp.where(better, cid, wacc)
                        macc = jnp.where(better, vs, macc)
                    gmv = bmin(macc)
                    gj = bmin(jnp.where(macc == gmv, wacc, bigi))
                    for jj in range(NACC):
                        upd = (jsel == jj) & lmask
                        ms[jj] = jnp.where(upd, gmv, ms[jj])
                        ws[jj] = jnp.where(upd, gj, ws[jj])
                o_v[r, :] = outv
                return carry1

            lax.fori_loop(0, RB8, row_body, 0)
            pltpu.sync_copy(o_v, idx_hbm.at[pl.ds(row0, RB8)])
            return carry0

        lax.fori_loop(0, RW // RB8, batch_body, 0)

    return k(dist)


def _knn(t):
    return _sc_topk(_dist(t))


def _proj_body(t_ref, wl_ref, wr_ref, hl_ref, hr_ref):
    t = t_ref[...]
    hl_ref[...] = jnp.dot(t, wl_ref[...], preferred_element_type=jnp.float32)
    hr_ref[...] = jnp.dot(t, wr_ref[...], preferred_element_type=jnp.float32)


def _proj(t, wl, wr):
    di, do = wl.shape
    RB = 512
    grid = N // RB
    return pl.pallas_call(
        _proj_body,
        grid=(grid,),
        in_specs=[
            pl.BlockSpec((RB, di), lambda i: (i, 0)),
            pl.BlockSpec((di, do), lambda i: (0, 0)),
            pl.BlockSpec((di, do), lambda i: (0, 0)),
        ],
        out_specs=[
            pl.BlockSpec((RB, do), lambda i: (i, 0)),
            pl.BlockSpec((RB, do), lambda i: (i, 0)),
        ],
        out_shape=[
            jax.ShapeDtypeStruct((N, do), jnp.float32),
            jax.ShapeDtypeStruct((N, do), jnp.float32),
        ],
    )(t, wl, wr)


def _sc_gather(table, idx_flat):
    D = table.shape[1]
    B = idx_flat.shape[0]
    NW = 32
    b_per_w = B // NW
    C = 128
    n_chunks = b_per_w // C
    mesh = plsc.VectorSubcoreMesh(core_axis_name="c", subcore_axis_name="s")

    @functools.partial(
        pl.kernel,
        mesh=mesh,
        out_type=jax.ShapeDtypeStruct((B, D), jnp.float32),
        scratch_types=[
            pltpu.VMEM((b_per_w,), jnp.int32),
            pltpu.VMEM((C, D), jnp.float32),
            pltpu.SemaphoreType.DMA,
        ],
    )
    def k(table_hbm, idx_hbm, out_hbm, idx_v, rows_v, sem):
        wid = lax.axis_index("s") * 2 + lax.axis_index("c")
        base = wid * b_per_w
        pltpu.sync_copy(idx_hbm.at[pl.ds(base, b_per_w)], idx_v)

        def body(c, carry):
            off = c * C
            pltpu.async_copy(
                table_hbm.at[idx_v.at[pl.ds(off, C)]], rows_v, sem
            ).wait()
            pltpu.sync_copy(rows_v, out_hbm.at[pl.ds(base + off, C)])
            return carry

        lax.fori_loop(0, n_chunks, body, 0)

    return k(table, idx_flat)


def _edge_body(hl_ref, hn_ref, a_ref, b_ref, out_ref):
    hl = hl_ref[...]
    hn = hn_ref[...]
    a = a_ref[...]
    z = hl[:, None, :] + hn
    z = jnp.where(z > 0, z, 0.2 * z)
    e = jnp.sum(z * a[None, :, :], axis=-1)
    e = e - jnp.max(e, axis=1, keepdims=True)
    w = jnp.exp(e)
    alpha = w / jnp.sum(w, axis=1, keepdims=True)
    out_ref[...] = jnp.sum(alpha[:, :, None] * hn, axis=1) + b_ref[...]


def _edge(hl, hn3, a, b):
    D = hl.shape[1]
    NB = 128
    grid = N // NB
    return pl.pallas_call(
        _edge_body,
        grid=(grid,),
        in_specs=[
            pl.BlockSpec((NB, D), lambda i: (i, 0)),
            pl.BlockSpec((NB, K, D), lambda i: (i, 0, 0)),
            pl.BlockSpec((1, D), lambda i: (0, 0)),
            pl.BlockSpec((1, D), lambda i: (0, 0)),
        ],
        out_specs=pl.BlockSpec((NB, D), lambda i: (i, 0)),
        out_shape=jax.ShapeDtypeStruct((N, D), jnp.float32),
    )(hl, hn3, a.reshape(1, D), b.reshape(1, D))


def _mean_body(a_ref, b_ref, o_ref):
    o_ref[...] = 0.5 * (a_ref[...] + b_ref[...])


def _mean2(a, b):
    RB = 1024
    return pl.pallas_call(
        _mean_body,
        grid=(N // RB,),
        in_specs=[
            pl.BlockSpec((RB, 256), lambda i: (i, 0)),
            pl.BlockSpec((RB, 256), lambda i: (i, 0)),
        ],
        out_specs=pl.BlockSpec((RB, 256), lambda i: (i, 0)),
        out_shape=jax.ShapeDtypeStruct((N, 256), jnp.float32),
    )(a, b)


def _mlp_body(y1_ref, y2_ref, y3_ref, y4_ref, w1_ref, b1_ref, w2_ref, b2_ref,
              w3_ref, b3_ref, out_ref):
    acc = jnp.dot(y1_ref[...], w1_ref[0:256, :],
                  preferred_element_type=jnp.float32)
    acc += jnp.dot(y2_ref[...], w1_ref[256:768, :],
                   preferred_element_type=jnp.float32)
    acc += jnp.dot(y3_ref[...], w1_ref[768:1280, :],
                   preferred_element_type=jnp.float32)
    acc += jnp.dot(y4_ref[...], w1_ref[1280:1792, :],
                   preferred_element_type=jnp.float32)
    h1 = jnp.maximum(acc + b1_ref[...], 0.0)
    h2 = jnp.maximum(
        jnp.dot(h1, w2_ref[...], preferred_element_type=jnp.float32)
        + b2_ref[...], 0.0)
    out_ref[...] = (
        jnp.dot(h2, w3_ref[...], preferred_element_type=jnp.float32)
        + b3_ref[...]
    )


def _mlp(y1, y2, y3, y4, w1, b1, w2, b2, w3, b3):
    RB = 512
    return pl.pallas_call(
        _mlp_body,
        grid=(N // RB,),
        in_specs=[
            pl.BlockSpec((RB, 256), lambda i: (i, 0)),
            pl.BlockSpec((RB, 512), lambda i: (i, 0)),
            pl.BlockSpec((RB, 512), lambda i: (i, 0)),
            pl.BlockSpec((RB, 512), lambda i: (i, 0)),
            pl.BlockSpec((1792, 256), lambda i: (0, 0)),
            pl.BlockSpec((1, 256), lambda i: (0, 0)),
            pl.BlockSpec((256, 64), lambda i: (0, 0)),
            pl.BlockSpec((1, 64), lambda i: (0, 0)),
            pl.BlockSpec((64, 3), lambda i: (0, 0)),
            pl.BlockSpec((1, 3), lambda i: (0, 0)),
        ],
        out_specs=pl.BlockSpec((RB, 3), lambda i: (i, 0)),
        out_shape=jax.ShapeDtypeStruct((N, 3), jnp.float32),
    )(y1, y2, y3, y4, w1, b1.reshape(1, 256), w2, b2.reshape(1, 64),
      w3, b3.reshape(1, 3))


def _gat_unit(t, wl, wr, a, b):
    do = wl.shape[1]
    idx = _knn(t)
    hl, hr = _proj(t, wl, wr)
    hn = _sc_gather(hr, idx.reshape(N * K))
    return _edge(hl, hn.reshape(N, K, do), a, b)


def kernel(x, ft_w0, ft_b0, ft_w1, ft_b1, g1_wl, g1_wr, g1_a, g1_b,
           g2_wl, g2_wr, g2_a, g2_b, g3_wl, g3_wr, g3_a, g3_b,
           g4_wl, g4_wr, g4_a, g4_b, mlp_w1, mlp_b1, mlp_w2, mlp_b2,
           mlp_w3, mlp_b3):
    xa = x[:, 0:3]
    xb = x[:, 3:9]
    f0, f1 = _stem(xa, xb, ft_w0, ft_b0, ft_w1, ft_b1)
    y0 = _gat_unit(f0, g1_wl, g1_wr, g1_a, g1_b)
    y1 = _gat_unit(f1, g1_wl, g1_wr, g1_a, g1_b)
    y = _mean2(y0, y1)
    y2 = _gat_unit(y, g2_wl, g2_wr, g2_a, g2_b)
    y3 = _gat_unit(y2, g3_wl, g3_wr, g3_a, g3_b)
    y4 = _gat_unit(y3, g4_wl, g4_wr, g4_a, g4_b)
    return _mlp(y, y2, y3, y4, mlp_w1, mlp_b1, mlp_w2, mlp_b2, mlp_w3, mlp_b3)


# speedup vs baseline: 5.8605x; 1.0002x over previous
import functools

import jax
import jax.numpy as jnp
from jax import lax
from jax.experimental import pallas as pl
from jax.experimental.pallas import tpu as pltpu
from jax.experimental.pallas import tpu_sc as plsc

N = 4096
K = 16
BIGF = 3.0e38


def _stem_body(xa_ref, xb_ref, w0_ref, b0_ref, w1_ref, b1_ref, f0_ref, f1_ref):
    f0_ref[...] = (
        jnp.dot(xa_ref[...], w0_ref[...], preferred_element_type=jnp.float32)
        + b0_ref[...]
    )
    f1_ref[...] = (
        jnp.dot(xb_ref[...], w1_ref[...], preferred_element_type=jnp.float32)
        + b1_ref[...]
    )


def _stem(xa, xb, w0, b0, w1, b1):
    RB = 1024
    grid = N // RB
    return pl.pallas_call(
        _stem_body,
        grid=(grid,),
        in_specs=[
            pl.BlockSpec((RB, 3), lambda i: (i, 0)),
            pl.BlockSpec((RB, 6), lambda i: (i, 0)),
            pl.BlockSpec((3, 256), lambda i: (0, 0)),
            pl.BlockSpec((1, 256), lambda i: (0, 0)),
            pl.BlockSpec((6, 256), lambda i: (0, 0)),
            pl.BlockSpec((1, 256), lambda i: (0, 0)),
        ],
        out_specs=[
            pl.BlockSpec((RB, 256), lambda i: (i, 0)),
            pl.BlockSpec((RB, 256), lambda i: (i, 0)),
        ],
        out_shape=[
            jax.ShapeDtypeStruct((N, 256), jnp.float32),
            jax.ShapeDtypeStruct((N, 256), jnp.float32),
        ],
    )(xa, xb, w0, b0.reshape(1, 256), w1, b1.reshape(1, 256))


def _dist_body(t_blk_ref, t_all_ref, dist_ref):
    rb = pl.program_id(0)
    RB = t_blk_ref.shape[0]
    t_blk = t_blk_ref[...]
    sq_blk = jnp.sum(t_blk * t_blk, axis=1)[:, None]

    CT = 1024
    for ct in range(N // CT):
        t_tile = t_all_ref[pl.ds(ct * CT, CT), :]
        sq_tile = jnp.sum(t_tile * t_tile, axis=1)[None, :]
        prod = lax.dot_general(
            t_blk, t_tile, (((1,), (1,)), ((), ())),
            preferred_element_type=jnp.float32,
        )
        d2 = sq_blk + sq_tile - 2.0 * prod
        col = ct * CT + lax.broadcasted_iota(jnp.int32, (RB, CT), 1)
        row = rb * RB + lax.broadcasted_iota(jnp.int32, (RB, CT), 0)
        dist_ref[:, pl.ds(ct * CT, CT)] = jnp.where(col == row, BIGF, d2)


def _dist(t):
    d = t.shape[1]
    RB = 256
    return pl.pallas_call(
        _dist_body,
        grid=(N // RB,),
        in_specs=[
            pl.BlockSpec((RB, d), lambda i: (i, 0)),
            pl.BlockSpec((N, d), lambda i: (0, 0)),
        ],
        out_specs=pl.BlockSpec((RB, N), lambda i: (i, 0)),
        out_shape=jax.ShapeDtypeStruct((N, N), jnp.float32),
    )(t, t)


def _sc_topk(dist):
    NW = 32
    RW = N // NW
    RB8 = 8
    NACC = 8
    CC = N // (16 * NACC)
    mesh = plsc.VectorSubcoreMesh(core_axis_name="c", subcore_axis_name="s")

    @functools.partial(
        pl.kernel,
        mesh=mesh,
        out_type=jax.ShapeDtypeStruct((N, K), jnp.int32),
        scratch_types=[
            pltpu.VMEM((RB8, N), jnp.float32),
            pltpu.VMEM((RB8, K), jnp.int32),
            pltpu.VMEM((16,), jnp.int32),
        ],
    )
    def k(dist_hbm, idx_hbm, d_v, o_v, sci_v):
        wid = lax.axis_index("s") * 2 + lax.axis_index("c")
        base = wid * RW
        lane = lax.broadcasted_iota(jnp.int32, (16,), 0)
        bigf = jnp.full((16,), BIGF, jnp.float32)
        bigi = jnp.full((16,), N, jnp.int32)

        xors = [(lane ^ st).reshape(16, 1) for st in (8, 4, 2, 1)]
        _dnums = lax.GatherDimensionNumbers(
            offset_dims=(), collapsed_slice_dims=(0,), start_index_map=(0,))

        def bmin(x):
            for xi in xors:
                perm = lax.gather(
                    x, xi, _dnums, (1,),
                    mode=lax.GatherScatterMode.PROMISE_IN_BOUNDS)
                x = jnp.minimum(x, perm)
            return x

        def batch_body(bb, carry0):
            row0 = base + bb * RB8
            pltpu.sync_copy(dist_hbm.at[pl.ds(row0, RB8)], d_v)

            def row_body(r, carry1):
                def tb(cc, mw):
                    ms, ws = list(mw[0]), list(mw[1])
                    for jj in range(NACC):
                        c = cc * NACC + jj
                        dv = d_v[r, pl.ds(c * 16, 16)]
                        cid = lane + c * 16
                        better = dv < ms[jj]
                        ws[jj] = jnp.where(better, cid, ws[jj])
                        ms[jj] = jnp.where(better, dv, ms[jj])
                    return (tuple(ms), tuple(ws))

                ms, ws = lax.fori_loop(
                    0, CC, tb,
                    (tuple(bigf for _ in range(NACC)),
                     tuple(bigi for _ in range(NACC))))
                ms, ws = list(ms), list(ws)

                outv = jnp.zeros((16,), jnp.int32)
                for kk in range(K):
                    t01 = jnp.minimum(ms[0], ms[1])
                    t23 = jnp.minimum(ms[2], ms[3])
                    t45 = jnp.minimum(ms[4], ms[5])
                    t67 = jnp.minimum(ms[6], ms[7])
                    tmin = jnp.minimum(jnp.minimum(t01, t23),
                                       jnp.minimum(t45, t67))
                    mv = bmin(tmin)
                    wc = bigi
                    for jj in range(NACC):
                        wc = jnp.where(ms[jj] == mv,
                                       jnp.minimum(wc, ws[jj]), wc)
                    j = bmin(wc)
                    outv = jnp.where(lane == kk, j, outv)
                    js = j[0]
                    cj = js >> 4
                    lstar = j & 15
                    lmask = lane == lstar
                    dvj = d_v[r, pl.ds(cj * 16, 16)]
                    d_v[r, pl.ds(cj * 16, 16)] = jnp.where(lmask, bigf, dvj)
                    if kk == K - 1:
                        continue
                    jsel = (j >> 4) & (NACC - 1)
                    gsel = (js >> 4) & (NACC - 1)
                    cbase = (jsel << 4) + lane
                    macc = bigf
                    wacc = bigi
                    for cc in range(32):
                        v = d_v[r, pl.ds(cc * 128 + gsel * 16, 16)]
                        cid = cbase + cc * 128
                        better = v < macc
                        wacc = jnp.where(better, cid, wacc)
                        macc = jnp.where(better, v, macc)
                    for jj in range(NACC):
                        upd = jsel == jj
                        ms[jj] = jnp.where(upd, macc, ms[jj])
                        ws[jj] = jnp.where(upd, wacc, ws[jj])
                o_v[r, :] = outv
                return carry1

            lax.fori_loop(0, RB8, row_body, 0)
            pltpu.sync_copy(o_v, idx_hbm.at[pl.ds(row0, RB8)])
            return carry0

        lax.fori_loop(0, RW // RB8, batch_body, 0)

    return k(dist)


def _knn(t):
    return _sc_topk(_dist(t))


def _proj_body(t_ref, wl_ref, wr_ref, hl_ref, hr_ref):
    t = t_ref[...]
    hl_ref[...] = jnp.dot(t, wl_ref[...], preferred_element_type=jnp.float32)
    hr_ref[...] = jnp.dot(t, wr_ref[...], preferred_element_type=jnp.float32)


def _proj(t, wl, wr):
    di, do = wl.shape
    RB = 512
    grid = N // RB
    return pl.pallas_call(
        _proj_body,
        grid=(grid,),
        in_specs=[
            pl.BlockSpec((RB, di), lambda i: (i, 0)),
            pl.BlockSpec((di, do), lambda i: (0, 0)),
            pl.BlockSpec((di, do), lambda i: (0, 0)),
        ],
        out_specs=[
            pl.BlockSpec((RB, do), lambda i: (i, 0)),
            pl.BlockSpec((RB, do), lambda i: (i, 0)),
        ],
        out_shape=[
            jax.ShapeDtypeStruct((N, do), jnp.float32),
            jax.ShapeDtypeStruct((N, do), jnp.float32),
        ],
    )(t, wl, wr)


def _sc_gather(table, idx_flat):
    D = table.shape[1]
    B = idx_flat.shape[0]
    NW = 32
    b_per_w = B // NW
    C = 128
    n_chunks = b_per_w // C
    mesh = plsc.VectorSubcoreMesh(core_axis_name="c", subcore_axis_name="s")

    @functools.partial(
        pl.kernel,
        mesh=mesh,
        out_type=jax.ShapeDtypeStruct((B, D), jnp.float32),
        scratch_types=[
            pltpu.VMEM((b_per_w,), jnp.int32),
            pltpu.VMEM((C, D), jnp.float32),
            pltpu.SemaphoreType.DMA,
        ],
    )
    def k(table_hbm, idx_hbm, out_hbm, idx_v, rows_v, sem):
        wid = lax.axis_index("s") * 2 + lax.axis_index("c")
        base = wid * b_per_w
        pltpu.sync_copy(idx_hbm.at[pl.ds(base, b_per_w)], idx_v)

        def body(c, carry):
            off = c * C
            pltpu.async_copy(
                table_hbm.at[idx_v.at[pl.ds(off, C)]], rows_v, sem
            ).wait()
            pltpu.sync_copy(rows_v, out_hbm.at[pl.ds(base + off, C)])
            return carry

        lax.fori_loop(0, n_chunks, body, 0)

    return k(table, idx_flat)


def _edge_body(hl_ref, hn_ref, a_ref, b_ref, out_ref):
    hl = hl_ref[...]
    hn = hn_ref[...]
    a = a_ref[...]
    z = hl[:, None, :] + hn
    z = jnp.where(z > 0, z, 0.2 * z)
    e = jnp.sum(z * a[None, :, :], axis=-1)
    e = e - jnp.max(e, axis=1, keepdims=True)
    w = jnp.exp(e)
    alpha = w / jnp.sum(w, axis=1, keepdims=True)
    out_ref[...] = jnp.sum(alpha[:, :, None] * hn, axis=1) + b_ref[...]


def _edge(hl, hn3, a, b):
    D = hl.shape[1]
    NB = 128
    grid = N // NB
    return pl.pallas_call(
        _edge_body,
        grid=(grid,),
        in_specs=[
            pl.BlockSpec((NB, D), lambda i: (i, 0)),
            pl.BlockSpec((NB, K, D), lambda i: (i, 0, 0)),
            pl.BlockSpec((1, D), lambda i: (0, 0)),
            pl.BlockSpec((1, D), lambda i: (0, 0)),
        ],
        out_specs=pl.BlockSpec((NB, D), lambda i: (i, 0)),
        out_shape=jax.ShapeDtypeStruct((N, D), jnp.float32),
    )(hl, hn3, a.reshape(1, D), b.reshape(1, D))


def _mean_body(a_ref, b_ref, o_ref):
    o_ref[...] = 0.5 * (a_ref[...] + b_ref[...])


def _mean2(a, b):
    RB = 1024
    return pl.pallas_call(
        _mean_body,
        grid=(N // RB,),
        in_specs=[
            pl.BlockSpec((RB, 256), lambda i: (i, 0)),
            pl.BlockSpec((RB, 256), lambda i: (i, 0)),
        ],
        out_specs=pl.BlockSpec((RB, 256), lambda i: (i, 0)),
        out_shape=jax.ShapeDtypeStruct((N, 256), jnp.float32),
    )(a, b)


def _mlp_body(y1_ref, y2_ref, y3_ref, y4_ref, w1_ref, b1_ref, w2_ref, b2_ref,
              w3_ref, b3_ref, out_ref):
    acc = jnp.dot(y1_ref[...], w1_ref[0:256, :],
                  preferred_element_type=jnp.float32)
    acc += jnp.dot(y2_ref[...], w1_ref[256:768, :],
                   preferred_element_type=jnp.float32)
    acc += jnp.dot(y3_ref[...], w1_ref[768:1280, :],
                   preferred_element_type=jnp.float32)
    acc += jnp.dot(y4_ref[...], w1_ref[1280:1792, :],
                   preferred_element_type=jnp.float32)
    h1 = jnp.maximum(acc + b1_ref[...], 0.0)
    h2 = jnp.maximum(
        jnp.dot(h1, w2_ref[...], preferred_element_type=jnp.float32)
        + b2_ref[...], 0.0)
    out_ref[...] = (
        jnp.dot(h2, w3_ref[...], preferred_element_type=jnp.float32)
        + b3_ref[...]
    )


def _mlp(y1, y2, y3, y4, w1, b1, w2, b2, w3, b3):
    RB = 512
    return pl.pallas_call(
        _mlp_body,
        grid=(N // RB,),
        in_specs=[
            pl.BlockSpec((RB, 256), lambda i: (i, 0)),
            pl.BlockSpec((RB, 512), lambda i: (i, 0)),
            pl.BlockSpec((RB, 512), lambda i: (i, 0)),
            pl.BlockSpec((RB, 512), lambda i: (i, 0)),
            pl.BlockSpec((1792, 256), lambda i: (0, 0)),
            pl.BlockSpec((1, 256), lambda i: (0, 0)),
            pl.BlockSpec((256, 64), lambda i: (0, 0)),
            pl.BlockSpec((1, 64), lambda i: (0, 0)),
            pl.BlockSpec((64, 3), lambda i: (0, 0)),
            pl.BlockSpec((1, 3), lambda i: (0, 0)),
        ],
        out_specs=pl.BlockSpec((RB, 3), lambda i: (i, 0)),
        out_shape=jax.ShapeDtypeStruct((N, 3), jnp.float32),
    )(y1, y2, y3, y4, w1, b1.reshape(1, 256), w2, b2.reshape(1, 64),
      w3, b3.reshape(1, 3))


def _gat_unit(t, wl, wr, a, b):
    do = wl.shape[1]
    idx = _knn(t)
    hl, hr = _proj(t, wl, wr)
    hn = _sc_gather(hr, idx.reshape(N * K))
    return _edge(hl, hn.reshape(N, K, do), a, b)


def kernel(x, ft_w0, ft_b0, ft_w1, ft_b1, g1_wl, g1_wr, g1_a, g1_b,
           g2_wl, g2_wr, g2_a, g2_b, g3_wl, g3_wr, g3_a, g3_b,
           g4_wl, g4_wr, g4_a, g4_b, mlp_w1, mlp_b1, mlp_w2, mlp_b2,
           mlp_w3, mlp_b3):
    xa = x[:, 0:3]
    xb = x[:, 3:9]
    f0, f1 = _stem(xa, xb, ft_w0, ft_b0, ft_w1, ft_b1)
    y0 = _gat_unit(f0, g1_wl, g1_wr, g1_a, g1_b)
    y1 = _gat_unit(f1, g1_wl, g1_wr, g1_a, g1_b)
    y = _mean2(y0, y1)
    y2 = _gat_unit(y, g2_wl, g2_wr, g2_a, g2_b)
    y3 = _gat_unit(y2, g3_wl, g3_wr, g3_a, g3_b)
    y4 = _gat_unit(y3, g4_wl, g4_wr, g4_a, g4_b)
    return _mlp(y, y2, y3, y4, mlp_w1, mlp_b1, mlp_w2, mlp_b2, mlp_w3, mlp_b3)


# speedup vs baseline: 5.8606x; 1.0000x over previous
import functools

import jax
import jax.numpy as jnp
from jax import lax
from jax.experimental import pallas as pl
from jax.experimental.pallas import tpu as pltpu
from jax.experimental.pallas import tpu_sc as plsc

N = 4096
K = 16
BIGF = 3.0e38


def _stem_body(xa_ref, xb_ref, w0_ref, b0_ref, w1_ref, b1_ref, f0_ref, f1_ref):
    f0_ref[...] = (
        jnp.dot(xa_ref[...], w0_ref[...], preferred_element_type=jnp.float32)
        + b0_ref[...]
    )
    f1_ref[...] = (
        jnp.dot(xb_ref[...], w1_ref[...], preferred_element_type=jnp.float32)
        + b1_ref[...]
    )


def _stem(xa, xb, w0, b0, w1, b1):
    RB = 1024
    grid = N // RB
    return pl.pallas_call(
        _stem_body,
        grid=(grid,),
        in_specs=[
            pl.BlockSpec((RB, 3), lambda i: (i, 0)),
            pl.BlockSpec((RB, 6), lambda i: (i, 0)),
            pl.BlockSpec((3, 256), lambda i: (0, 0)),
            pl.BlockSpec((1, 256), lambda i: (0, 0)),
            pl.BlockSpec((6, 256), lambda i: (0, 0)),
            pl.BlockSpec((1, 256), lambda i: (0, 0)),
        ],
        out_specs=[
            pl.BlockSpec((RB, 256), lambda i: (i, 0)),
            pl.BlockSpec((RB, 256), lambda i: (i, 0)),
        ],
        out_shape=[
            jax.ShapeDtypeStruct((N, 256), jnp.float32),
            jax.ShapeDtypeStruct((N, 256), jnp.float32),
        ],
    )(xa, xb, w0, b0.reshape(1, 256), w1, b1.reshape(1, 256))


def _dist_body(t_blk_ref, t_all_ref, dist_ref):
    rb = pl.program_id(0)
    RB = t_blk_ref.shape[0]
    t_blk = t_blk_ref[...]
    sq_blk = jnp.sum(t_blk * t_blk, axis=1)[:, None]

    CT = 1024
    for ct in range(N // CT):
        t_tile = t_all_ref[pl.ds(ct * CT, CT), :]
        sq_tile = jnp.sum(t_tile * t_tile, axis=1)[None, :]
        prod = lax.dot_general(
            t_blk, t_tile, (((1,), (1,)), ((), ())),
            preferred_element_type=jnp.float32,
        )
        d2 = sq_blk + sq_tile - 2.0 * prod
        col = ct * CT + lax.broadcasted_iota(jnp.int32, (RB, CT), 1)
        row = rb * RB + lax.broadcasted_iota(jnp.int32, (RB, CT), 0)
        dist_ref[:, pl.ds(ct * CT, CT)] = jnp.where(col == row, BIGF, d2)


def _dist(t):
    d = t.shape[1]
    RB = 256
    return pl.pallas_call(
        _dist_body,
        grid=(N // RB,),
        in_specs=[
            pl.BlockSpec((RB, d), lambda i: (i, 0)),
            pl.BlockSpec((N, d), lambda i: (0, 0)),
        ],
        out_specs=pl.BlockSpec((RB, N), lambda i: (i, 0)),
        out_shape=jax.ShapeDtypeStruct((N, N), jnp.float32),
    )(t, t)


def _sc_topk(dist):
    NW = 32
    RW = N // NW
    RB8 = 8
    NACC = 8
    CC = N // (16 * NACC)
    mesh = plsc.VectorSubcoreMesh(core_axis_name="c", subcore_axis_name="s")

    @functools.partial(
        pl.kernel,
        mesh=mesh,
        out_type=jax.ShapeDtypeStruct((N, K), jnp.int32),
        scratch_types=[
            pltpu.VMEM((RB8, N), jnp.float32),
            pltpu.VMEM((RB8, K), jnp.int32),
        ],
    )
    def k(dist_hbm, idx_hbm, d_v, o_v):
        wid = lax.axis_index("s") * 2 + lax.axis_index("c")
        base = wid * RW
        lane = lax.broadcasted_iota(jnp.int32, (16,), 0)
        bigf = jnp.full((16,), BIGF, jnp.float32)
        bigi = jnp.full((16,), N, jnp.int32)

        xors = [(lane ^ st).reshape(16, 1) for st in (8, 4, 2, 1)]
        _dnums = lax.GatherDimensionNumbers(
            offset_dims=(), collapsed_slice_dims=(0,), start_index_map=(0,))

        def bmin(x):
            for xi in xors:
                perm = lax.gather(
                    x, xi, _dnums, (1,),
                    mode=lax.GatherScatterMode.PROMISE_IN_BOUNDS)
                x = jnp.minimum(x, perm)
            return x

        def batch_body(bb, carry0):
            row0 = base + bb * RB8
            pltpu.sync_copy(dist_hbm.at[pl.ds(row0, RB8)], d_v)

            def row_body(r, carry1):
                def tb(cc, mw):
                    ms, ws = list(mw[0]), list(mw[1])
                    for jj in range(NACC):
                        c = cc * NACC + jj
                        dv = d_v[r, pl.ds(c * 16, 16)]
                        cid = lane + c * 16
                        better = dv < ms[jj]
                        ws[jj] = jnp.where(better, cid, ws[jj])
                        ms[jj] = jnp.where(better, dv, ms[jj])
                    return (tuple(ms), tuple(ws))

                ms, ws = lax.fori_loop(
                    0, CC, tb,
                    (tuple(bigf for _ in range(NACC)),
                     tuple(bigi for _ in range(NACC))))
                ms, ws = list(ms), list(ws)

                outv = jnp.zeros((16,), jnp.int32)
                for kk in range(K):
                    t01 = jnp.minimum(ms[0], ms[1])
                    t23 = jnp.minimum(ms[2], ms[3])
                    t45 = jnp.minimum(ms[4], ms[5])
                    t67 = jnp.minimum(ms[6], ms[7])
                    tmin = jnp.minimum(jnp.minimum(t01, t23),
                                       jnp.minimum(t45, t67))
                    mv = bmin(tmin)
                    wc = bigi
                    for jj in range(NACC):
                        wc = jnp.where(ms[jj] == mv,
                                       jnp.minimum(wc, ws[jj]), wc)
                    j = bmin(wc)
                    outv = jnp.where(lane == kk, j, outv)
                    js = j[0]
                    cj = js >> 4
                    lstar = j & 15
                    lmask = lane == lstar
                    dvj = d_v[r, pl.ds(cj * 16, 16)]
                    d_v[r, pl.ds(cj * 16, 16)] = jnp.where(lmask, bigf, dvj)
                    if kk == K - 1:
                        continue
                    jsel = (j >> 4) & (NACC - 1)
                    gsel = (js >> 4) & (NACC - 1)
                    cbase = (jsel << 4) + lane
                    macc = bigf
                    wacc = bigi
                    for cc in range(32):
                        v = d_v[r, pl.ds(cc * 128 + gsel * 16, 16)]
                        cid = cbase + cc * 128
                        better = v < macc
                        wacc = jnp.where(better, cid, wacc)
                        macc = jnp.where(better, v, macc)
                    for jj in range(NACC):
                        upd = jsel == jj
                        ms[jj] = jnp.where(upd, macc, ms[jj])
                        ws[jj] = jnp.where(upd, wacc, ws[jj])
                o_v[r, :] = outv
                return carry1

            lax.fori_loop(0, RB8, row_body, 0)
            pltpu.sync_copy(o_v, idx_hbm.at[pl.ds(row0, RB8)])
            return carry0

        lax.fori_loop(0, RW // RB8, batch_body, 0)

    return k(dist)


def _knn(t):
    return _sc_topk(_dist(t))


def _proj_body(t_ref, wl_ref, wr_ref, hl_ref, hr_ref):
    t = t_ref[...]
    hl_ref[...] = jnp.dot(t, wl_ref[...], preferred_element_type=jnp.float32)
    hr_ref[...] = jnp.dot(t, wr_ref[...], preferred_element_type=jnp.float32)


def _proj(t, wl, wr):
    di, do = wl.shape
    RB = 512
    grid = N // RB
    return pl.pallas_call(
        _proj_body,
        grid=(grid,),
        in_specs=[
            pl.BlockSpec((RB, di), lambda i: (i, 0)),
            pl.BlockSpec((di, do), lambda i: (0, 0)),
            pl.BlockSpec((di, do), lambda i: (0, 0)),
        ],
        out_specs=[
            pl.BlockSpec((RB, do), lambda i: (i, 0)),
            pl.BlockSpec((RB, do), lambda i: (i, 0)),
        ],
        out_shape=[
            jax.ShapeDtypeStruct((N, do), jnp.float32),
            jax.ShapeDtypeStruct((N, do), jnp.float32),
        ],
    )(t, wl, wr)


def _sc_gather(table, idx_flat):
    D = table.shape[1]
    B = idx_flat.shape[0]
    NW = 32
    b_per_w = B // NW
    C = 128
    n_chunks = b_per_w // C
    mesh = plsc.VectorSubcoreMesh(core_axis_name="c", subcore_axis_name="s")

    @functools.partial(
        pl.kernel,
        mesh=mesh,
        out_type=jax.ShapeDtypeStruct((B, D), jnp.float32),
        scratch_types=[
            pltpu.VMEM((b_per_w,), jnp.int32),
            pltpu.VMEM((C, D), jnp.float32),
            pltpu.SemaphoreType.DMA,
        ],
    )
    def k(table_hbm, idx_hbm, out_hbm, idx_v, rows_v, sem):
        wid = lax.axis_index("s") * 2 + lax.axis_index("c")
        base = wid * b_per_w
        pltpu.sync_copy(idx_hbm.at[pl.ds(base, b_per_w)], idx_v)

        def body(c, carry):
            off = c * C
            pltpu.async_copy(
                table_hbm.at[idx_v.at[pl.ds(off, C)]], rows_v, sem
            ).wait()
            pltpu.sync_copy(rows_v, out_hbm.at[pl.ds(base + off, C)])
            return carry

        lax.fori_loop(0, n_chunks, body, 0)

    return k(table, idx_flat)


def _edge_body(hl_ref, hn_ref, a_ref, b_ref, out_ref):
    hl = hl_ref[...]
    hn = hn_ref[...]
    a = a_ref[...]
    z = hl[:, None, :] + hn
    z = jnp.where(z > 0, z, 0.2 * z)
    e = jnp.sum(z * a[None, :, :], axis=-1)
    e = e - jnp.max(e, axis=1, keepdims=True)
    w = jnp.exp(e)
    alpha = w / jnp.sum(w, axis=1, keepdims=True)
    out_ref[...] = jnp.sum(alpha[:, :, None] * hn, axis=1) + b_ref[...]


def _edge(hl, hn3, a, b):
    D = hl.shape[1]
    NB = 128
    grid = N // NB
    return pl.pallas_call(
        _edge_body,
        grid=(grid,),
        in_specs=[
            pl.BlockSpec((NB, D), lambda i: (i, 0)),
            pl.BlockSpec((NB, K, D), lambda i: (i, 0, 0)),
            pl.BlockSpec((1, D), lambda i: (0, 0)),
            pl.BlockSpec((1, D), lambda i: (0, 0)),
        ],
        out_specs=pl.BlockSpec((NB, D), lambda i: (i, 0)),
        out_shape=jax.ShapeDtypeStruct((N, D), jnp.float32),
    )(hl, hn3, a.reshape(1, D), b.reshape(1, D))


def _mean_body(a_ref, b_ref, o_ref):
    o_ref[...] = 0.5 * (a_ref[...] + b_ref[...])


def _mean2(a, b):
    RB = 1024
    return pl.pallas_call(
        _mean_body,
        grid=(N // RB,),
        in_specs=[
            pl.BlockSpec((RB, 256), lambda i: (i, 0)),
            pl.BlockSpec((RB, 256), lambda i: (i, 0)),
        ],
        out_specs=pl.BlockSpec((RB, 256), lambda i: (i, 0)),
        out_shape=jax.ShapeDtypeStruct((N, 256), jnp.float32),
    )(a, b)


def _mlp_body(y1_ref, y2_ref, y3_ref, y4_ref, w1_ref, b1_ref, w2_ref, b2_ref,
              w3_ref, b3_ref, out_ref):
    acc = jnp.dot(y1_ref[...], w1_ref[0:256, :],
                  preferred_element_type=jnp.float32)
    acc += jnp.dot(y2_ref[...], w1_ref[256:768, :],
                   preferred_element_type=jnp.float32)
    acc += jnp.dot(y3_ref[...], w1_ref[768:1280, :],
                   preferred_element_type=jnp.float32)
    acc += jnp.dot(y4_ref[...], w1_ref[1280:1792, :],
                   preferred_element_type=jnp.float32)
    h1 = jnp.maximum(acc + b1_ref[...], 0.0)
    h2 = jnp.maximum(
        jnp.dot(h1, w2_ref[...], preferred_element_type=jnp.float32)
        + b2_ref[...], 0.0)
    out_ref[...] = (
        jnp.dot(h2, w3_ref[...], preferred_element_type=jnp.float32)
        + b3_ref[...]
    )


def _mlp(y1, y2, y3, y4, w1, b1, w2, b2, w3, b3):
    RB = 512
    return pl.pallas_call(
        _mlp_body,
        grid=(N // RB,),
        in_specs=[
            pl.BlockSpec((RB, 256), lambda i: (i, 0)),
            pl.BlockSpec((RB, 512), lambda i: (i, 0)),
            pl.BlockSpec((RB, 512), lambda i: (i, 0)),
            pl.BlockSpec((RB, 512), lambda i: (i, 0)),
            pl.BlockSpec((1792, 256), lambda i: (0, 0)),
            pl.BlockSpec((1, 256), lambda i: (0, 0)),
            pl.BlockSpec((256, 64), lambda i: (0, 0)),
            pl.BlockSpec((1, 64), lambda i: (0, 0)),
            pl.BlockSpec((64, 3), lambda i: (0, 0)),
            pl.BlockSpec((1, 3), lambda i: (0, 0)),
        ],
        out_specs=pl.BlockSpec((RB, 3), lambda i: (i, 0)),
        out_shape=jax.ShapeDtypeStruct((N, 3), jnp.float32),
    )(y1, y2, y3, y4, w1, b1.reshape(1, 256), w2, b2.reshape(1, 64),
      w3, b3.reshape(1, 3))


def _gat_unit(t, wl, wr, a, b):
    do = wl.shape[1]
    idx = _knn(t)
    hl, hr = _proj(t, wl, wr)
    hn = _sc_gather(hr, idx.reshape(N * K))
    return _edge(hl, hn.reshape(N, K, do), a, b)


def kernel(x, ft_w0, ft_b0, ft_w1, ft_b1, g1_wl, g1_wr, g1_a, g1_b,
           g2_wl, g2_wr, g2_a, g2_b, g3_wl, g3_wr, g3_a, g3_b,
           g4_wl, g4_wr, g4_a, g4_b, mlp_w1, mlp_b1, mlp_w2, mlp_b2,
           mlp_w3, mlp_b3):
    xa = x[:, 0:3]
    xb = x[:, 3:9]
    f0, f1 = _stem(xa, xb, ft_w0, ft_b0, ft_w1, ft_b1)
    y0 = _gat_unit(f0, g1_wl, g1_wr, g1_a, g1_b)
    y1 = _gat_unit(f1, g1_wl, g1_wr, g1_a, g1_b)
    y = _mean2(y0, y1)
    y2 = _gat_unit(y, g2_wl, g2_wr, g2_a, g2_b)
    y3 = _gat_unit(y2, g3_wl, g3_wr, g3_a, g3_b)
    y4 = _gat_unit(y3, g4_wl, g4_wr, g4_a, g4_b)
    return _mlp(y, y2, y3, y4, mlp_w1, mlp_b1, mlp_w2, mlp_b2, mlp_w3, mlp_b3)
